# Initial kernel scaffold; baseline (speedup 1.0000x reference)
import jax
import jax.numpy as jnp
from jax.experimental import pallas as pl


def kernel(x, edge_index, edge_weight, W1_w, W1_b, W2_w, W2_b):
    raise NotImplementedError("write your pallas kernel here")


# speedup vs baseline: 3.8318x; 3.8318x over previous
import functools

import jax
import jax.numpy as jnp
from jax import lax
from jax.experimental import pallas as pl
from jax.experimental.pallas import tpu as pltpu
from jax.experimental.pallas import tpu_sc as plsc

N_NODES = 10000
D = 128
E_EDGES = 320000

NC = 2
NS = 16
NW = NC * NS
L = 16

CH = 128
CHUNKS = 79
EW_PER = CH * CHUNKS
E_PAD = EW_PER * NW
N_PAD = 10240
ROWS_PER_TILE = N_PAD // NS


def _sc_mesh():
    return plsc.VectorSubcoreMesh(core_axis_name="c", subcore_axis_name="s")


def _worker_id():
    return lax.axis_index("s") * NC + lax.axis_index("c")


def _rowsum_body(dst_hbm, ew_hbm, zz_hbm, p0_hbm, p1_hbm,
                 dst_v, ew_v, rs_sh):
    cid = lax.axis_index("c")
    sid = lax.axis_index("s")
    wid = _worker_id()

    pltpu.sync_copy(zz_hbm.at[pl.ds(0, ROWS_PER_TILE)],
                    rs_sh.at[pl.ds(sid * ROWS_PER_TILE, ROWS_PER_TILE)])
    plsc.subcore_barrier()

    def chunk(c, carry):
        base = wid * EW_PER + c * CH
        pltpu.sync_copy(dst_hbm.at[pl.ds(base, CH)], dst_v)
        pltpu.sync_copy(ew_hbm.at[pl.ds(base, CH)], ew_v)
        pltpu.sync_copy(ew_v, rs_sh.at[dst_v], add=True)
        return carry

    lax.fori_loop(0, CHUNKS, chunk, 0)
    plsc.subcore_barrier()

    sl = pl.ds(sid * ROWS_PER_TILE, ROWS_PER_TILE)

    @pl.when(cid == 0)
    def _():
        pltpu.sync_copy(rs_sh.at[sl], p0_hbm.at[sl])

    @pl.when(cid == 1)
    def _():
        pltpu.sync_copy(rs_sh.at[sl], p1_hbm.at[sl])


def _rowsum_call(dst, ew, zz):
    kern = pl.kernel(
        _rowsum_body,
        out_type=(jax.ShapeDtypeStruct((N_PAD,), jnp.float32),
                  jax.ShapeDtypeStruct((N_PAD,), jnp.float32)),
        mesh=_sc_mesh(),
        scratch_types=[
            pltpu.VMEM((CH,), jnp.int32),
            pltpu.VMEM((CH,), jnp.float32),
            pltpu.VMEM_SHARED((N_PAD,), jnp.float32),
        ],
    )
    return kern(dst, ew, zz)


def _scale_rows(rows, wn_v):
    def egroup(g, carry):
        wv = wn_v[pl.ds(g * L, L)]
        for i in range(L):
            w = lax.gather(
                wv, jnp.full((L, 1), i, jnp.int32),
                lax.GatherDimensionNumbers(
                    offset_dims=(), collapsed_slice_dims=(0,),
                    start_index_map=(0,)),
                (1,), mode=lax.GatherScatterMode.PROMISE_IN_BOUNDS)
            e = g * L + i
            for j in range(D // L):
                sl = pl.ds(j * L, L)
                rows[e, sl] = rows[e, sl] * w
        return carry

    lax.fori_loop(0, CH // L, egroup, 0)


def _acc_writeout(cid, sid, acc_sh, a0_hbm, a1_hbm):
    sl = pl.ds(sid * ROWS_PER_TILE, ROWS_PER_TILE)

    @pl.when(cid == 0)
    def _():
        pltpu.sync_copy(acc_sh.at[sl], a0_hbm.at[sl])

    @pl.when(cid == 1)
    def _():
        pltpu.sync_copy(acc_sh.at[sl], a1_hbm.at[sl])


def _spmm1_body(x_hbm, src_hbm, dst_hbm, ew_hbm, p0_hbm, p1_hbm, zzr_hbm,
                a0_hbm, a1_hbm, wn_hbm,
                src_v, dst_v, ew_v, rs0_v, rs1_v, wn_v, rows, acc_sh, sem):
    cid = lax.axis_index("c")
    sid = lax.axis_index("s")
    wid = _worker_id()

    pltpu.sync_copy(zzr_hbm.at[pl.ds(0, ROWS_PER_TILE)],
                    acc_sh.at[pl.ds(sid * ROWS_PER_TILE, ROWS_PER_TILE)])
    plsc.subcore_barrier()

    def chunk(c, carry):
        base = wid * EW_PER + c * CH
        pltpu.sync_copy(src_hbm.at[pl.ds(base, CH)], src_v)
        pltpu.sync_copy(dst_hbm.at[pl.ds(base, CH)], dst_v)
        pltpu.sync_copy(ew_hbm.at[pl.ds(base, CH)], ew_v)
        pltpu.async_copy(x_hbm.at[src_v], rows, sem).wait()
        pltpu.async_copy(p0_hbm.at[dst_v], rs0_v, sem).wait()
        pltpu.async_copy(p1_hbm.at[dst_v], rs1_v, sem).wait()
        for j in range(CH // L):
            sl = pl.ds(j * L, L)
            rs = rs0_v[sl] + rs1_v[sl]
            wn_v[sl] = ew_v[sl] / jnp.maximum(rs, 1e-12)
        pltpu.sync_copy(wn_v, wn_hbm.at[pl.ds(base, CH)])
        _scale_rows(rows, wn_v)
        pltpu.sync_copy(rows, acc_sh.at[dst_v], add=True)
        return carry

    lax.fori_loop(0, CHUNKS, chunk, 0)
    plsc.subcore_barrier()
    _acc_writeout(cid, sid, acc_sh, a0_hbm, a1_hbm)


def _spmm2_body(x_hbm, src_hbm, dst_hbm, wn_hbm, zzr_hbm,
                a0_hbm, a1_hbm,
                src_v, dst_v, wn_v, rows, acc_sh, sem):
    cid = lax.axis_index("c")
    sid = lax.axis_index("s")
    wid = _worker_id()

    pltpu.sync_copy(zzr_hbm.at[pl.ds(0, ROWS_PER_TILE)],
                    acc_sh.at[pl.ds(sid * ROWS_PER_TILE, ROWS_PER_TILE)])
    plsc.subcore_barrier()

    def chunk(c, carry):
        base = wid * EW_PER + c * CH
        pltpu.sync_copy(src_hbm.at[pl.ds(base, CH)], src_v)
        pltpu.sync_copy(dst_hbm.at[pl.ds(base, CH)], dst_v)
        pltpu.sync_copy(wn_hbm.at[pl.ds(base, CH)], wn_v)
        pltpu.async_copy(x_hbm.at[src_v], rows, sem).wait()
        _scale_rows(rows, wn_v)
        pltpu.sync_copy(rows, acc_sh.at[dst_v], add=True)
        return carry

    lax.fori_loop(0, CHUNKS, chunk, 0)
    plsc.subcore_barrier()
    _acc_writeout(cid, sid, acc_sh, a0_hbm, a1_hbm)


def _spmm1_call(x_pad, src, dst, ew, p0, p1, zzr):
    kern = pl.kernel(
        _spmm1_body,
        out_type=(jax.ShapeDtypeStruct((N_PAD, D), jnp.float32),
                  jax.ShapeDtypeStruct((N_PAD, D), jnp.float32),
                  jax.ShapeDtypeStruct((E_PAD,), jnp.float32)),
        mesh=_sc_mesh(),
        scratch_types=[
            pltpu.VMEM((CH,), jnp.int32),
            pltpu.VMEM((CH,), jnp.int32),
            pltpu.VMEM((CH,), jnp.float32),
            pltpu.VMEM((CH,), jnp.float32),
            pltpu.VMEM((CH,), jnp.float32),
            pltpu.VMEM((CH,), jnp.float32),
            pltpu.VMEM((CH, D), jnp.float32),
            pltpu.VMEM_SHARED((N_PAD, D), jnp.float32),
            pltpu.SemaphoreType.DMA,
        ],
    )
    return kern(x_pad, src, dst, ew, p0, p1, zzr)


def _spmm2_call(h_pad, src, dst, wn, zzr):
    kern = pl.kernel(
        _spmm2_body,
        out_type=(jax.ShapeDtypeStruct((N_PAD, D), jnp.float32),
                  jax.ShapeDtypeStruct((N_PAD, D), jnp.float32)),
        mesh=_sc_mesh(),
        scratch_types=[
            pltpu.VMEM((CH,), jnp.int32),
            pltpu.VMEM((CH,), jnp.int32),
            pltpu.VMEM((CH,), jnp.float32),
            pltpu.VMEM((CH, D), jnp.float32),
            pltpu.VMEM_SHARED((N_PAD, D), jnp.float32),
            pltpu.SemaphoreType.DMA,
        ],
    )
    return kern(h_pad, src, dst, wn, zzr)


BM = 512


def _mlp_body(x_ref, a0_ref, a1_ref, w_ref, b_ref, o_ref, *, relu):
    hn = a0_ref[...] + a1_ref[...]
    wx = w_ref[:, :D]
    wn = w_ref[:, D:]
    z = (lax.dot_general(x_ref[...], wx, (((1,), (1,)), ((), ())),
                         preferred_element_type=jnp.float32)
         + lax.dot_general(hn, wn, (((1,), (1,)), ((), ())),
                           preferred_element_type=jnp.float32)
         + b_ref[...])
    if relu:
        z = jnp.maximum(z, 0.0)
    o_ref[...] = z


def _mlp_call(x_pad, a0, a1, w, b2d, relu):
    grid = N_PAD // BM
    return pl.pallas_call(
        functools.partial(_mlp_body, relu=relu),
        grid=(grid,),
        in_specs=[
            pl.BlockSpec((BM, D), lambda i: (i, 0)),
            pl.BlockSpec((BM, D), lambda i: (i, 0)),
            pl.BlockSpec((BM, D), lambda i: (i, 0)),
            pl.BlockSpec((D, 2 * D), lambda i: (0, 0)),
            pl.BlockSpec((1, D), lambda i: (0, 0)),
        ],
        out_specs=pl.BlockSpec((BM, D), lambda i: (i, 0)),
        out_shape=jax.ShapeDtypeStruct((N_PAD, D), jnp.float32),
    )(x_pad, a0, a1, w, b2d)


def kernel(x, edge_index, edge_weight, W1_w, W1_b, W2_w, W2_b):
    dst = edge_index[0]
    src = edge_index[1]

    pad_e = E_PAD - E_EDGES
    src_p = jnp.concatenate(
        [src, jnp.full((pad_e,), N_NODES, jnp.int32)])
    dst_p = jnp.concatenate(
        [dst, jnp.full((pad_e,), N_NODES, jnp.int32)])
    ew_p = jnp.concatenate([edge_weight, jnp.zeros((pad_e,), jnp.float32)])

    x_pad = jnp.zeros((N_PAD, D), jnp.float32).at[:N_NODES].set(x)
    zzr = jnp.zeros((ROWS_PER_TILE, D), jnp.float32)
    zz1 = jnp.zeros((ROWS_PER_TILE,), jnp.float32)

    p0, p1 = _rowsum_call(dst_p, ew_p, zz1)
    a0, a1, wn = _spmm1_call(x_pad, src_p, dst_p, ew_p, p0, p1, zzr)
    h_pad = _mlp_call(x_pad, a0, a1, W1_w, W1_b.reshape(1, D), relu=True)
    q0, q1 = _spmm2_call(h_pad, src_p, dst_p, wn, zzr)
    h2 = _mlp_call(h_pad, q0, q1, W2_w, W2_b.reshape(1, D), relu=False)
    return h2[:N_NODES]
